# baseline (speedup 1.0000x reference)
import functools
import jax
import jax.numpy as jnp
from jax import lax
from jax.experimental import pallas as pl
from jax.experimental.pallas import tpu as pltpu
from jax.experimental.pallas import tpu_sc as plsc

BATCH = 16384
N_UNIT = 1024
N_BIN = 50
N_PAD = 64
NC = 2
NS = 16
NW = NC * NS
N_COLG = 8
N_ROWQ = NW // N_COLG
U_PER_W = N_UNIT // N_COLG
B_PER_W = BATCH // N_ROWQ
CHUNK = 256
L = 16


def _sc_body(in_hbm, xsp_hbm, ys_hbm, out_hbm, xs_v, ys_v, inb, outb):
    wid = lax.axis_index("s") * NC + lax.axis_index("c")
    u0 = (wid // N_ROWQ) * U_PER_W
    r0_base = (wid % N_ROWQ) * B_PER_W

    pltpu.sync_copy(xsp_hbm.at[pl.ds(u0 * N_PAD, U_PER_W * N_PAD)], xs_v)
    pltpu.sync_copy(ys_hbm.at[pl.ds(u0 * N_BIN, U_PER_W * N_BIN)], ys_v)

    lane = lax.iota(jnp.int32, L)
    halves = []
    for h in range(U_PER_W // L):
        uvec = lane + (h * L)
        xbase = uvec * N_PAD
        ybase = uvec * N_BIN
        xs_first = plsc.load_gather(xs_v, [xbase])
        xs_last = plsc.load_gather(xs_v, [xbase + (N_BIN - 1)])
        ys_first = plsc.load_gather(ys_v, [ybase])
        ys_last = plsc.load_gather(ys_v, [ybase + (N_BIN - 1)])
        halves.append((xbase, ybase, xs_first, xs_last, ys_first, ys_last))

    def row_body(row, _):
        for h, (xbase, ybase, xs_first, xs_last, ys_first, ys_last) in enumerate(halves):
            x = inb[row, pl.ds(h * L, L)]
            rf = xbase
            for step in (32, 16, 8, 4, 2, 1):
                probe = plsc.load_gather(xs_v, [rf + (step - 1)])
                rf = jnp.where(probe <= x, rf + step, rf)
            r = rf - xbase
            lo = jnp.clip(r, 1, N_BIN - 1) - 1
            xlo_i = xbase + lo
            ylo_i = ybase + lo
            x_lo = plsc.load_gather(xs_v, [xlo_i])
            x_hi = plsc.load_gather(xs_v, [xlo_i + 1])
            y_lo = plsc.load_gather(ys_v, [ylo_i])
            y_hi = plsc.load_gather(ys_v, [ylo_i + 1])
            t = (x - x_lo) / jnp.maximum(x_hi - x_lo, jnp.float32(1e-12))
            res = y_lo + t * (y_hi - y_lo)
            res = jnp.where(x <= xs_first, ys_first,
                            jnp.where(x >= xs_last, ys_last, res))
            outb[row, pl.ds(h * L, L)] = res
        return _

    def chunk_body(i, _):
        row0 = r0_base + i * CHUNK
        pltpu.sync_copy(in_hbm.at[pl.ds(row0, CHUNK), pl.ds(u0, U_PER_W)], inb)
        lax.fori_loop(0, CHUNK, row_body, None, unroll=2)
        pltpu.sync_copy(outb, out_hbm.at[pl.ds(row0, CHUNK), pl.ds(u0, U_PER_W)])
        return _

    lax.fori_loop(0, B_PER_W // CHUNK, chunk_body, None)


@jax.jit
def kernel(inputs, xs, ys):
    xs_pad = jnp.pad(xs, ((0, 0), (0, N_PAD - N_BIN)),
                     constant_values=jnp.finfo(jnp.float32).max)
    mesh = plsc.VectorSubcoreMesh(core_axis_name="c", subcore_axis_name="s")
    sc = pl.kernel(
        _sc_body,
        out_type=jax.ShapeDtypeStruct((BATCH, N_UNIT), jnp.float32),
        mesh=mesh,
        scratch_types=[
            pltpu.VMEM((U_PER_W * N_PAD,), jnp.float32),
            pltpu.VMEM((U_PER_W * N_BIN,), jnp.float32),
            pltpu.VMEM((CHUNK, U_PER_W), jnp.float32),
            pltpu.VMEM((CHUNK, U_PER_W), jnp.float32),
        ],
        compiler_params=pltpu.CompilerParams(needs_layout_passes=False),
    )
    return sc(inputs, xs_pad.reshape(-1), ys.reshape(-1))


# speedup vs baseline: 2.6577x; 2.6577x over previous
import functools
import jax
import jax.numpy as jnp
from jax import lax
from jax.experimental import pallas as pl
from jax.experimental.pallas import tpu as pltpu
from jax.experimental.pallas import tpu_sc as plsc

BATCH = 16384
N_UNIT = 1024
N_BIN = 50
N_PAD = 64
NC = 2
NS = 16
NW = NC * NS
N_COLG = 8
N_ROWQ = NW // N_COLG
U_PER_W = N_UNIT // N_COLG
B_PER_W = BATCH // N_ROWQ
CHUNK = 256
L = 16


def _sc_body(in_hbm, xsp_hbm, ys_hbm, out_hbm, xs_v, ys_v, inb, outb):
    wid = lax.axis_index("s") * NC + lax.axis_index("c")
    u0 = (wid // N_ROWQ) * U_PER_W
    r0_base = (wid % N_ROWQ) * B_PER_W

    pltpu.sync_copy(xsp_hbm.at[pl.ds(u0 * N_PAD, U_PER_W * N_PAD)], xs_v)
    pltpu.sync_copy(ys_hbm.at[pl.ds(u0 * N_BIN, U_PER_W * N_BIN)], ys_v)

    lane = lax.iota(jnp.int32, L)
    halves = []
    for h in range(U_PER_W // L):
        uvec = lane + (h * L)
        xbase = uvec * N_PAD
        ybase = uvec * N_BIN
        xs_first = plsc.load_gather(xs_v, [xbase])
        xs_last = plsc.load_gather(xs_v, [xbase + (N_BIN - 1)])
        ys_first = plsc.load_gather(ys_v, [ybase])
        ys_last = plsc.load_gather(ys_v, [ybase + (N_BIN - 1)])
        halves.append((xbase, ybase, xs_first, xs_last, ys_first, ys_last))

    def row_body(row):
        for h, (xbase, ybase, xs_first, xs_last, ys_first, ys_last) in enumerate(halves):
            x = inb[row, pl.ds(h * L, L)]
            rf = xbase
            for step in (32, 16, 8, 4, 2, 1):
                probe = plsc.load_gather(xs_v, [rf + (step - 1)])
                rf = jnp.where(probe <= x, rf + step, rf)
            r = rf - xbase
            lo = jnp.clip(r, 1, N_BIN - 1) - 1
            xlo_i = xbase + lo
            ylo_i = ybase + lo
            x_lo = plsc.load_gather(xs_v, [xlo_i])
            x_hi = plsc.load_gather(xs_v, [xlo_i + 1])
            y_lo = plsc.load_gather(ys_v, [ylo_i])
            y_hi = plsc.load_gather(ys_v, [ylo_i + 1])
            t = (x - x_lo) / jnp.maximum(x_hi - x_lo, jnp.float32(1e-12))
            res = y_lo + t * (y_hi - y_lo)
            res = jnp.where(x <= xs_first, ys_first,
                            jnp.where(x >= xs_last, ys_last, res))
            outb[row, pl.ds(h * L, L)] = res

    def chunk_body(i, _):
        row0 = r0_base + i * CHUNK
        pltpu.sync_copy(in_hbm.at[pl.ds(row0, CHUNK), pl.ds(u0, U_PER_W)], inb)
        plsc.parallel_loop(0, CHUNK, step=1, unroll=2)(row_body)
        pltpu.sync_copy(outb, out_hbm.at[pl.ds(row0, CHUNK), pl.ds(u0, U_PER_W)])
        return _

    lax.fori_loop(0, B_PER_W // CHUNK, chunk_body, None)


@jax.jit
def kernel(inputs, xs, ys):
    xs_pad = jnp.pad(xs, ((0, 0), (0, N_PAD - N_BIN)),
                     constant_values=jnp.finfo(jnp.float32).max)
    mesh = plsc.VectorSubcoreMesh(core_axis_name="c", subcore_axis_name="s")
    sc = pl.kernel(
        _sc_body,
        out_type=jax.ShapeDtypeStruct((BATCH, N_UNIT), jnp.float32),
        mesh=mesh,
        scratch_types=[
            pltpu.VMEM((U_PER_W * N_PAD,), jnp.float32),
            pltpu.VMEM((U_PER_W * N_BIN,), jnp.float32),
            pltpu.VMEM((CHUNK, U_PER_W), jnp.float32),
            pltpu.VMEM((CHUNK, U_PER_W), jnp.float32),
        ],
        compiler_params=pltpu.CompilerParams(needs_layout_passes=False),
    )
    return sc(inputs, xs_pad.reshape(-1), ys.reshape(-1))
